# speedup vs baseline: 25.8910x; 25.8910x over previous
import functools
import math

import jax
import jax.numpy as jnp
from jax import lax
from jax.experimental import pallas as pl
from jax.experimental.pallas import tpu as pltpu
from jax.experimental.pallas import tpu_sc as plsc

B, L = 1024, 200
EMBED = 128
HOUR_EMBED = EMBED // 4
N = B * L

NC, NS = 2, 16
NW = NC * NS
NPW = N // NW
CHUNK = 128
NCHUNK = NPW // CHUNK


def _make_sc_gather():
    mesh = plsc.VectorSubcoreMesh(core_axis_name="c", subcore_axis_name="s")

    @functools.partial(
        pl.kernel,
        mesh=mesh,
        out_type=jax.ShapeDtypeStruct((N, EMBED), jnp.float32),
        scratch_types=[
            pltpu.VMEM((NCHUNK, CHUNK), jnp.int32),
            pltpu.VMEM((CHUNK, EMBED), jnp.float32),
            pltpu.VMEM((CHUNK, EMBED), jnp.float32),
            pltpu.SemaphoreType.DMA,
            pltpu.SemaphoreType.DMA,
            pltpu.SemaphoreType.DMA,
            pltpu.SemaphoreType.DMA,
        ],
    )
    def sc_gather(table_hbm, idx_hbm, out_hbm, idx_v, rows0, rows1,
                  gsem0, gsem1, osem0, osem1):
        wid = lax.axis_index("s") * NC + lax.axis_index("c")
        base = wid * NPW
        pltpu.sync_copy(idx_hbm.at[wid], idx_v)

        rows = (rows0, rows1)
        gsem = (gsem0, gsem1)
        osem = (osem0, osem1)

        pltpu.async_copy(table_hbm.at[idx_v.at[0]], rows0, gsem0)
        pltpu.async_copy(table_hbm.at[idx_v.at[1]], rows1, gsem1)

        def body(jj, _):
            for bsel in range(2):
                j = jj * 2 + bsel
                pltpu.make_async_copy(
                    table_hbm.at[idx_v.at[j]], rows[bsel], gsem[bsel]
                ).wait()
                pltpu.async_copy(
                    rows[bsel],
                    out_hbm.at[pl.ds(base + j * CHUNK, CHUNK)],
                    osem[bsel],
                )

                @pl.when(j + 2 < NCHUNK)
                def _():
                    pltpu.make_async_copy(
                        rows[bsel],
                        out_hbm.at[pl.ds(base + j * CHUNK, CHUNK)],
                        osem[bsel],
                    ).wait()
                    pltpu.async_copy(
                        table_hbm.at[idx_v.at[j + 2]], rows[bsel], gsem[bsel]
                    )

            return 0

        lax.fori_loop(0, NCHUNK // 2, body, 0)
        pltpu.make_async_copy(
            rows0, out_hbm.at[pl.ds(base + (NCHUNK - 2) * CHUNK, CHUNK)], osem0
        ).wait()
        pltpu.make_async_copy(
            rows1, out_hbm.at[pl.ds(base + (NCHUNK - 1) * CHUNK, CHUNK)], osem1
        ).wait()

    return sc_gather


_sc_gather = _make_sc_gather()


def _tc_body(tok_ref, hour_ref, ht_ref, w1_ref, w2_ref, peb_ref, out_ref):
    bb = tok_ref.shape[0]
    tok = tok_ref[...].reshape(bb * L, EMBED)
    hour = hour_ref[...]
    hcb = jnp.dot(ht_ref[...], w2_ref[...], preferred_element_type=jnp.float32)
    onehot = (hour[..., None] == lax.broadcasted_iota(jnp.int32, (bb, L, 32), 2))
    onehot = onehot.astype(jnp.float32).reshape(bb * L, 32)
    acc = jnp.dot(tok, w1_ref[...], preferred_element_type=jnp.float32)
    acc = acc + jnp.dot(onehot, hcb, preferred_element_type=jnp.float32)
    acc = acc.reshape(bb, L, EMBED)
    acc = acc + peb_ref[...][None, :, :]
    out_ref[...] = jnp.tanh(acc)


def _tc_fuse(token_emb, hour_seq, hour_table, w1, w2, peb):
    BB = 32
    grid = (B // BB,)
    return pl.pallas_call(
        _tc_body,
        grid=grid,
        in_specs=[
            pl.BlockSpec((BB, L, EMBED), lambda i: (i, 0, 0)),
            pl.BlockSpec((BB, L), lambda i: (i, 0)),
            pl.BlockSpec((32, HOUR_EMBED), lambda i: (0, 0)),
            pl.BlockSpec((EMBED, EMBED), lambda i: (0, 0)),
            pl.BlockSpec((HOUR_EMBED, EMBED), lambda i: (0, 0)),
            pl.BlockSpec((L, EMBED), lambda i: (0, 0)),
        ],
        out_specs=pl.BlockSpec((BB, L, EMBED), lambda i: (i, 0, 0)),
        out_shape=jax.ShapeDtypeStruct((B, L, EMBED), jnp.float32),
    )(token_emb, hour_seq, hour_table, w1, w2, peb)


def _sinusoidal_pe(seq_len, d_model):
    pos = jnp.arange(seq_len, dtype=jnp.float32)[:, None]
    div_term = jnp.exp(
        jnp.arange(0, d_model, 2, dtype=jnp.float32)
        * (-math.log(10000.0) / d_model)
    )
    pe = jnp.zeros((seq_len, d_model), dtype=jnp.float32)
    pe = pe.at[:, 0::2].set(jnp.sin(pos * div_term))
    pe = pe.at[:, 1::2].set(jnp.cos(pos * div_term))
    return pe


def kernel(token_seq, hour_seq, loc_table, hour_table, W, b):
    idx = token_seq.astype(jnp.int32).reshape(NW, NCHUNK, CHUNK)
    token_emb = _sc_gather(loc_table, idx)
    token_emb = token_emb.reshape(B, L, EMBED)
    peb = _sinusoidal_pe(L, EMBED) + b[None, :]
    out = _tc_fuse(
        token_emb,
        hour_seq.astype(jnp.int32),
        hour_table,
        W[:EMBED],
        W[EMBED:],
        peb,
    )
    return out


# speedup vs baseline: 26.0825x; 1.0074x over previous
import functools
import math

import jax
import jax.numpy as jnp
from jax import lax
from jax.experimental import pallas as pl
from jax.experimental.pallas import tpu as pltpu
from jax.experimental.pallas import tpu_sc as plsc

B, L = 1024, 200
EMBED = 128
HOUR_EMBED = EMBED // 4
N = B * L

NC, NS = 2, 16
NW = NC * NS
NPW = N // NW
CHUNK = 128
NCHUNK = NPW // CHUNK


NBUF = 4


def _make_sc_gather():
    mesh = plsc.VectorSubcoreMesh(core_axis_name="c", subcore_axis_name="s")

    @functools.partial(
        pl.kernel,
        mesh=mesh,
        out_type=jax.ShapeDtypeStruct((N, EMBED), jnp.float32),
        scratch_types=[
            pltpu.VMEM((NCHUNK, CHUNK), jnp.int32),
        ]
        + [pltpu.VMEM((CHUNK, EMBED), jnp.float32) for _ in range(NBUF)]
        + [pltpu.SemaphoreType.DMA for _ in range(2 * NBUF)],
    )
    def sc_gather(table_hbm, idx_hbm, out_hbm, idx_v, *bufs):
        rows = bufs[:NBUF]
        gsem = bufs[NBUF : 2 * NBUF]
        osem = bufs[2 * NBUF : 3 * NBUF]
        wid = lax.axis_index("s") * NC + lax.axis_index("c")
        base = wid * NPW
        pltpu.sync_copy(idx_hbm.at[wid], idx_v)

        def out_at(j):
            return out_hbm.at[pl.ds(base + j * CHUNK, CHUNK)]

        for b in range(NBUF - 1):
            pltpu.async_copy(table_hbm.at[idx_v.at[b]], rows[b], gsem[b])

        def body(jj, _):
            for b in range(NBUF):
                j = jj * NBUF + b
                pb = (b - 1) % NBUF
                nb = (b + NBUF - 1) % NBUF

                @pl.when(j >= 1)
                def _():
                    pltpu.make_async_copy(
                        rows[pb], out_at(j - 1), osem[pb]
                    ).wait()

                @pl.when(j + NBUF - 1 < NCHUNK)
                def _():
                    pltpu.async_copy(
                        table_hbm.at[idx_v.at[j + NBUF - 1]], rows[nb], gsem[nb]
                    )

                pltpu.make_async_copy(
                    table_hbm.at[idx_v.at[j]], rows[b], gsem[b]
                ).wait()
                pltpu.async_copy(rows[b], out_at(j), osem[b])
            return 0

        lax.fori_loop(0, NCHUNK // NBUF, body, 0, unroll=False)
        for j in range(NCHUNK - NCHUNK % NBUF, NCHUNK):
            b = j % NBUF
            pltpu.make_async_copy(rows[(b - 1) % NBUF], out_at(j - 1),
                                  osem[(b - 1) % NBUF]).wait()
            pltpu.make_async_copy(
                table_hbm.at[idx_v.at[j]], rows[b], gsem[b]
            ).wait()
            pltpu.async_copy(rows[b], out_at(j), osem[b])
        jl = NCHUNK - 1
        pltpu.make_async_copy(rows[jl % NBUF], out_at(jl), osem[jl % NBUF]).wait()

    return sc_gather


_sc_gather = _make_sc_gather()


def _tc_body(tok_ref, hour_ref, ht_ref, w1_ref, w2_ref, peb_ref, out_ref):
    bb = tok_ref.shape[0]
    tok = tok_ref[...].reshape(bb * L, EMBED)
    hour = hour_ref[...]
    hcb = jnp.dot(ht_ref[...], w2_ref[...], preferred_element_type=jnp.float32)
    onehot = (hour[..., None] == lax.broadcasted_iota(jnp.int32, (bb, L, 32), 2))
    onehot = onehot.astype(jnp.float32).reshape(bb * L, 32)
    acc = jnp.dot(tok, w1_ref[...], preferred_element_type=jnp.float32)
    acc = acc + jnp.dot(onehot, hcb, preferred_element_type=jnp.float32)
    acc = acc.reshape(bb, L, EMBED)
    acc = acc + peb_ref[...][None, :, :]
    out_ref[...] = jnp.tanh(acc)


def _tc_fuse(token_emb, hour_seq, hour_table, w1, w2, peb):
    BB = 32
    grid = (B // BB,)
    return pl.pallas_call(
        _tc_body,
        grid=grid,
        in_specs=[
            pl.BlockSpec((BB, L, EMBED), lambda i: (i, 0, 0)),
            pl.BlockSpec((BB, L), lambda i: (i, 0)),
            pl.BlockSpec((32, HOUR_EMBED), lambda i: (0, 0)),
            pl.BlockSpec((EMBED, EMBED), lambda i: (0, 0)),
            pl.BlockSpec((HOUR_EMBED, EMBED), lambda i: (0, 0)),
            pl.BlockSpec((L, EMBED), lambda i: (0, 0)),
        ],
        out_specs=pl.BlockSpec((BB, L, EMBED), lambda i: (i, 0, 0)),
        out_shape=jax.ShapeDtypeStruct((B, L, EMBED), jnp.float32),
    )(token_emb, hour_seq, hour_table, w1, w2, peb)


def _sinusoidal_pe(seq_len, d_model):
    pos = jnp.arange(seq_len, dtype=jnp.float32)[:, None]
    div_term = jnp.exp(
        jnp.arange(0, d_model, 2, dtype=jnp.float32)
        * (-math.log(10000.0) / d_model)
    )
    pe = jnp.zeros((seq_len, d_model), dtype=jnp.float32)
    pe = pe.at[:, 0::2].set(jnp.sin(pos * div_term))
    pe = pe.at[:, 1::2].set(jnp.cos(pos * div_term))
    return pe


def kernel(token_seq, hour_seq, loc_table, hour_table, W, b):
    idx = token_seq.astype(jnp.int32).reshape(NW, NCHUNK, CHUNK)
    token_emb = _sc_gather(loc_table, idx)
    token_emb = token_emb.reshape(B, L, EMBED)
    peb = _sinusoidal_pe(L, EMBED) + b[None, :]
    out = _tc_fuse(
        token_emb,
        hour_seq.astype(jnp.int32),
        hour_table,
        W[:EMBED],
        W[EMBED:],
        peb,
    )
    return out


# speedup vs baseline: 26.4605x; 1.0145x over previous
import functools
import math

import jax
import jax.numpy as jnp
from jax import lax
from jax.experimental import pallas as pl
from jax.experimental.pallas import tpu as pltpu
from jax.experimental.pallas import tpu_sc as plsc

B, L = 1024, 200
EMBED = 128
HOUR_EMBED = EMBED // 4
N = B * L

NSPLIT = 2
BH = B // NSPLIT
NH = N // NSPLIT

NC, NS = 2, 16
NW = NC * NS
NPW = NH // NW
CHUNK = 128
NCHUNK = NPW // CHUNK
NBUF = 4


def _make_sc_gather():
    mesh = plsc.VectorSubcoreMesh(core_axis_name="c", subcore_axis_name="s")

    @functools.partial(
        pl.kernel,
        mesh=mesh,
        out_type=jax.ShapeDtypeStruct((NH, EMBED), jnp.float32),
        scratch_types=[
            pltpu.VMEM((NCHUNK, CHUNK), jnp.int32),
        ]
        + [pltpu.VMEM((CHUNK, EMBED), jnp.float32) for _ in range(NBUF)]
        + [pltpu.SemaphoreType.DMA for _ in range(2 * NBUF)],
    )
    def sc_gather(table_hbm, idx_hbm, out_hbm, idx_v, *bufs):
        rows = bufs[:NBUF]
        gsem = bufs[NBUF : 2 * NBUF]
        osem = bufs[2 * NBUF : 3 * NBUF]
        wid = lax.axis_index("s") * NC + lax.axis_index("c")
        base = wid * NPW
        pltpu.sync_copy(idx_hbm.at[wid], idx_v)

        def out_at(j):
            return out_hbm.at[pl.ds(base + j * CHUNK, CHUNK)]

        for b in range(NBUF - 1):
            pltpu.async_copy(table_hbm.at[idx_v.at[b]], rows[b], gsem[b])

        def body(jj, _):
            for b in range(NBUF):
                j = jj * NBUF + b
                pb = (b - 1) % NBUF
                nb = (b + NBUF - 1) % NBUF

                @pl.when(j >= 1)
                def _():
                    pltpu.make_async_copy(
                        rows[pb], out_at(j - 1), osem[pb]
                    ).wait()

                @pl.when(j + NBUF - 1 < NCHUNK)
                def _():
                    pltpu.async_copy(
                        table_hbm.at[idx_v.at[j + NBUF - 1]], rows[nb], gsem[nb]
                    )

                pltpu.make_async_copy(
                    table_hbm.at[idx_v.at[j]], rows[b], gsem[b]
                ).wait()
                pltpu.async_copy(rows[b], out_at(j), osem[b])
            return 0

        lax.fori_loop(0, NCHUNK // NBUF, body, 0, unroll=False)
        for j in range(NCHUNK - NCHUNK % NBUF, NCHUNK):
            b = j % NBUF
            pltpu.make_async_copy(rows[(b - 1) % NBUF], out_at(j - 1),
                                  osem[(b - 1) % NBUF]).wait()
            pltpu.make_async_copy(
                table_hbm.at[idx_v.at[j]], rows[b], gsem[b]
            ).wait()
            pltpu.async_copy(rows[b], out_at(j), osem[b])
        jl = NCHUNK - 1
        pltpu.make_async_copy(rows[jl % NBUF], out_at(jl), osem[jl % NBUF]).wait()

    return sc_gather


_sc_gather = _make_sc_gather()

BB = 32


def _tc_body(tok_ref, hour_ref, ht_ref, w1_ref, w2_ref, peb_ref, out_ref):
    tok = tok_ref[...].reshape(BB * L, EMBED)
    hour = hour_ref[...]
    hcb = jnp.dot(ht_ref[...], w2_ref[...], preferred_element_type=jnp.float32)
    onehot = (hour[..., None] == lax.broadcasted_iota(jnp.int32, (BB, L, 32), 2))
    onehot = onehot.astype(jnp.float32).reshape(BB * L, 32)
    acc = jnp.dot(tok, w1_ref[...], preferred_element_type=jnp.float32)
    acc = acc + jnp.dot(onehot, hcb, preferred_element_type=jnp.float32)
    acc = acc.reshape(BB, L, EMBED)
    acc = acc + peb_ref[...][None, :, :]
    out_ref[...] = jnp.tanh(acc)


def _tc_body2(dst_ref, *rest):
    del dst_ref
    _tc_body(*rest)


_TC_COMMON_SPECS = [
    pl.BlockSpec((BB, L), lambda i: (i, 0)),
    pl.BlockSpec((32, HOUR_EMBED), lambda i: (0, 0)),
    pl.BlockSpec((EMBED, EMBED), lambda i: (0, 0)),
    pl.BlockSpec((HOUR_EMBED, EMBED), lambda i: (0, 0)),
    pl.BlockSpec((L, EMBED), lambda i: (0, 0)),
]


def _tc_fuse_first(tok, hour, ht, w1, w2, peb):
    return pl.pallas_call(
        _tc_body,
        grid=(BH // BB,),
        in_specs=[pl.BlockSpec((BB, L, EMBED), lambda i: (i, 0, 0))]
        + _TC_COMMON_SPECS,
        out_specs=pl.BlockSpec((BB, L, EMBED), lambda i: (i, 0, 0)),
        out_shape=jax.ShapeDtypeStruct((B, L, EMBED), jnp.float32),
    )(tok, hour, ht, w1, w2, peb)


def _tc_fuse_second(dst, tok, hour, ht, w1, w2, peb):
    off = BH // BB
    return pl.pallas_call(
        _tc_body2,
        grid=(BH // BB,),
        in_specs=[
            pl.BlockSpec(memory_space=pl.ANY),
            pl.BlockSpec((BB, L, EMBED), lambda i: (i, 0, 0)),
        ]
        + _TC_COMMON_SPECS,
        out_specs=pl.BlockSpec((BB, L, EMBED), lambda i: (i + off, 0, 0)),
        out_shape=jax.ShapeDtypeStruct((B, L, EMBED), jnp.float32),
        input_output_aliases={0: 0},
    )(dst, tok, hour, ht, w1, w2, peb)


def _sinusoidal_pe(seq_len, d_model):
    pos = jnp.arange(seq_len, dtype=jnp.float32)[:, None]
    div_term = jnp.exp(
        jnp.arange(0, d_model, 2, dtype=jnp.float32)
        * (-math.log(10000.0) / d_model)
    )
    pe = jnp.zeros((seq_len, d_model), dtype=jnp.float32)
    pe = pe.at[:, 0::2].set(jnp.sin(pos * div_term))
    pe = pe.at[:, 1::2].set(jnp.cos(pos * div_term))
    return pe


def kernel(token_seq, hour_seq, loc_table, hour_table, W, b):
    idx = token_seq.astype(jnp.int32).reshape(NSPLIT, NW, NCHUNK, CHUNK)
    hour = hour_seq.astype(jnp.int32)
    peb = _sinusoidal_pe(L, EMBED) + b[None, :]
    w1 = W[:EMBED]
    w2 = W[EMBED:]

    tok0 = _sc_gather(loc_table, idx[0]).reshape(BH, L, EMBED)
    tok1 = _sc_gather(loc_table, idx[1]).reshape(BH, L, EMBED)
    out = _tc_fuse_first(tok0, hour[:BH], hour_table, w1, w2, peb)
    out = _tc_fuse_second(out, tok1, hour[BH:], hour_table, w1, w2, peb)
    return out
